# baseline (speedup 1.0000x reference)
import functools

import jax
import jax.numpy as jnp
from jax import lax
from jax.experimental import pallas as pl
from jax.experimental.pallas import tpu as pltpu
from jax.experimental.pallas import tpu_sc as plsc

_BATCH = 4096
_HIST = 200
_HISTP = 256
_RING = 208
_DIM = 64
_NCLASS = 100

_NCHUNK = 13
_CHUNK = 16
_NGRP = _DIM // 16
_PAIR = 2 * _DIM


def _chunk_start(j):
  return min(16 * j, _HIST - _CHUNK)


def _sc_gather_sum(pair2, hoff2, table2):
  mesh = plsc.VectorSubcoreMesh(core_axis_name="c", subcore_axis_name="s")
  nw = mesh.num_cores * mesh.num_subcores
  rows_per_w = _BATCH // nw
  half = rows_per_w // 2

  @functools.partial(
      pl.kernel,
      out_type=jax.ShapeDtypeStruct((_BATCH, _PAIR), jnp.float32),
      mesh=mesh,
      scratch_types=[
          pltpu.VMEM((half, _HISTP), jnp.int32),
          pltpu.VMEM((half, _HISTP), jnp.int32),
          pltpu.VMEM((2, _RING, _PAIR), jnp.float32),
          pltpu.VMEM((rows_per_w, _PAIR), jnp.float32),
          pltpu.SemaphoreType.DMA,
          pltpu.SemaphoreType.DMA,
      ],
      compiler_params=pltpu.CompilerParams(use_tc_tiling_on_sc=True),
  )
  def k(pair_hbm, hoff_hbm, table_hbm, out_hbm, idx_v, hoff_v, rows_v, acc_v,
        sem0, sem1):
    wid = lax.axis_index("s") * mesh.num_cores + lax.axis_index("c")
    base = wid * rows_per_w

    def fire(bh, par, sem):
      for j in range(_NCHUNK):
        s = _chunk_start(j)
        iv = idx_v[bh, pl.ds(s, _CHUNK)]
        pltpu.async_copy(table_hbm.at[iv],
                         rows_v.at[par, pl.ds(s, _CHUNK)], sem)

    def drain(bh, par, sem):
      for j in range(_NCHUNK):
        s = _chunk_start(j)
        iv = idx_v[bh, pl.ds(s, _CHUNK)]
        pltpu.make_async_copy(
            table_hbm.at[iv], rows_v.at[par, pl.ds(s, _CHUNK)], sem).wait()

    def accumulate_and_store(b, bh, par):
      accs = tuple(jnp.zeros((16,), jnp.float32) for _ in range(_NGRP))

      def body(kk, accs):
        accs = list(accs)
        hv = hoff_v[bh, pl.ds(kk * 16, 16)]
        for r in range(16):
          row = kk * 16 + r
          off = hv[r]
          for g in range(_NGRP):
            accs[g] = accs[g] + rows_v[par, row, pl.ds(off + g * 16, 16)]
        return tuple(accs)

      accs = list(lax.fori_loop(0, _HIST // 16, body, accs))
      hv = hoff_v[bh, pl.ds(_HIST - 16, 16)]
      for r in range(8, 16):
        row = _HIST - 16 + r
        off = hv[r]
        for g in range(_NGRP):
          accs[g] = accs[g] + rows_v[par, row, pl.ds(off + g * 16, 16)]

      for g in range(_NGRP):
        acc_v[b, pl.ds(g * 16, 16)] = accs[g]

    for h in range(2):
      hbase = base + h * half
      pltpu.sync_copy(pair_hbm.at[pl.ds(hbase, half)], idx_v)
      pltpu.sync_copy(hoff_hbm.at[pl.ds(hbase, half)], hoff_v)

      fire(0, 0, sem0)

      def step(bb, _, h=h):
        b0 = 2 * bb
        b1 = 2 * bb + 1
        fire(b1, 1, sem1)
        drain(b0, 0, sem0)
        accumulate_and_store(h * half + b0, b0, 0)

        @pl.when(bb < half // 2 - 1)
        def _():
          fire(b0 + 2, 0, sem0)

        drain(b1, 1, sem1)
        accumulate_and_store(h * half + b1, b1, 1)
        return 0

      lax.fori_loop(0, half // 2, step, 0)

    pltpu.sync_copy(acc_v, out_hbm.at[pl.ds(base, rows_per_w)])

  return k(pair2, hoff2, table2)


def _tc_linear(sums2, fc_weight, fc_bias2):
  def body(x_ref, w_ref, b_ref, o_ref):
    acc = lax.dot_general(
        x_ref[:, :_DIM], w_ref[:, :],
        dimension_numbers=(((1,), (1,)), ((), ())),
        preferred_element_type=jnp.float32,
    )
    o_ref[:, :] = acc * (1.0 / _HIST) + b_ref[:, :]

  return pl.pallas_call(
      body,
      out_shape=jax.ShapeDtypeStruct((_BATCH, _NCLASS), jnp.float32),
  )(sums2, fc_weight, fc_bias2)


def kernel(text, embed_table, fc_weight, fc_bias):
  ti = text.astype(jnp.int32)
  pad = ((0, 0), (0, _HISTP - _HIST))
  pair2 = jnp.pad(ti >> 1, pad)
  hoff2 = jnp.pad((ti & 1) << 6, pad)
  table2 = embed_table.reshape(500000, _PAIR)
  sums2 = _sc_gather_sum(pair2, hoff2, table2)
  return _tc_linear(sums2, fc_weight, fc_bias.reshape(1, _NCLASS))


# speedup vs baseline: 1.6195x; 1.6195x over previous
import functools

import jax
import jax.numpy as jnp
from jax import lax
from jax.experimental import pallas as pl
from jax.experimental.pallas import tpu as pltpu
from jax.experimental.pallas import tpu_sc as plsc

_BATCH = 4096
_HIST = 200
_HISTP = 256
_RING = 208
_VOCAB = 1000000
_DIM = 64
_NCLASS = 100
_PROJ = 128

_NCHUNK = 13
_CHUNK = 16
_PGRP = _PROJ // 16

_MM_BLK = 2048


def _tc_project(table_t, fc_weight):
  def body(t_ref, w_ref, o_ref):
    acc = lax.dot_general(
        t_ref[:, :], w_ref[:, :],
        dimension_numbers=(((0,), (1,)), ((), ())),
        preferred_element_type=jnp.float32,
    )
    o_ref[:, :] = jnp.pad(acc, ((0, 0), (0, _PROJ - _NCLASS)))

  return pl.pallas_call(
      body,
      grid=((_VOCAB + _MM_BLK - 1) // _MM_BLK,),
      in_specs=[
          pl.BlockSpec((_DIM, _MM_BLK), lambda i: (0, i)),
          pl.BlockSpec((_NCLASS, _DIM), lambda i: (0, 0)),
      ],
      out_specs=pl.BlockSpec((_MM_BLK, _PROJ), lambda i: (i, 0)),
      out_shape=jax.ShapeDtypeStruct((_VOCAB, _PROJ), jnp.float32),
  )(table_t, fc_weight)


def _sc_gather_sum(idx2, proj):
  mesh = plsc.VectorSubcoreMesh(core_axis_name="c", subcore_axis_name="s")
  nw = mesh.num_cores * mesh.num_subcores
  rows_per_w = _BATCH // nw
  half = rows_per_w // 2

  @functools.partial(
      pl.kernel,
      out_type=jax.ShapeDtypeStruct((_BATCH, _PROJ), jnp.float32),
      mesh=mesh,
      scratch_types=[
          pltpu.VMEM((half, _HISTP), jnp.int32),
          pltpu.VMEM((2, _RING, _PROJ), jnp.float32),
          pltpu.VMEM((rows_per_w, _PROJ), jnp.float32),
          pltpu.SemaphoreType.DMA,
          pltpu.SemaphoreType.DMA,
      ],
      compiler_params=pltpu.CompilerParams(use_tc_tiling_on_sc=True),
  )
  def k(idx_hbm, proj_hbm, out_hbm, idx_v, rows_v, acc_v, sem0, sem1):
    wid = lax.axis_index("s") * mesh.num_cores + lax.axis_index("c")
    base = wid * rows_per_w

    starts = [min(16 * j, _HIST - _CHUNK) for j in range(_NCHUNK)]

    def fire(bh, par, sem):
      for s in starts:
        iv = idx_v[bh, pl.ds(s, _CHUNK)]
        pltpu.async_copy(proj_hbm.at[iv], rows_v.at[par, pl.ds(s, _CHUNK)],
                         sem)

    def drain(bh, par, sem):
      for s in starts:
        iv = idx_v[bh, pl.ds(s, _CHUNK)]
        pltpu.make_async_copy(
            proj_hbm.at[iv], rows_v.at[par, pl.ds(s, _CHUNK)], sem).wait()

    def accumulate_and_store(b, par):
      accs = tuple(jnp.zeros((16,), jnp.float32) for _ in range(_PGRP))

      def body(kk, accs):
        accs = list(accs)
        for r in range(8):
          row = kk * 8 + r
          for g in range(_PGRP):
            accs[g] = accs[g] + rows_v[par, row, pl.ds(g * 16, 16)]
        return tuple(accs)

      accs = lax.fori_loop(0, _HIST // 8, body, accs)
      for g in range(_PGRP):
        acc_v[b, pl.ds(g * 16, 16)] = accs[g]

    for h in range(2):
      hbase = base + h * half
      pltpu.sync_copy(idx_hbm.at[pl.ds(hbase, half)], idx_v)

      fire(0, 0, sem0)

      def step(bb, _, h=h):
        b0 = 2 * bb
        b1 = 2 * bb + 1
        fire(b1, 1, sem1)
        drain(b0, 0, sem0)
        accumulate_and_store(h * half + b0, 0)

        @pl.when(bb < half // 2 - 1)
        def _():
          fire(b0 + 2, 0, sem0)

        drain(b1, 1, sem1)
        accumulate_and_store(h * half + b1, 1)
        return 0

      lax.fori_loop(0, half // 2, step, 0)

    pltpu.sync_copy(acc_v, out_hbm.at[pl.ds(base, rows_per_w)])

  return k(idx2, proj)


def _tc_finish(sums2, fc_bias2):
  def body(x_ref, b_ref, o_ref):
    o_ref[:, :] = x_ref[:, :_NCLASS] * (1.0 / _HIST) + b_ref[:, :]

  return pl.pallas_call(
      body,
      out_shape=jax.ShapeDtypeStruct((_BATCH, _NCLASS), jnp.float32),
  )(sums2, fc_bias2)


def kernel(text, embed_table, fc_weight, fc_bias):
  idx2 = jnp.pad(text.astype(jnp.int32), ((0, 0), (0, _HISTP - _HIST)))
  proj = _tc_project(embed_table.T, fc_weight)
  sums2 = _sc_gather_sum(idx2, proj)
  return _tc_finish(sums2, fc_bias.reshape(1, _NCLASS))


# speedup vs baseline: 1.6196x; 1.0001x over previous
import functools

import jax
import jax.numpy as jnp
from jax import lax
from jax.experimental import pallas as pl
from jax.experimental.pallas import tpu as pltpu
from jax.experimental.pallas import tpu_sc as plsc

_BATCH = 4096
_HIST = 200
_HISTP = 256
_RING = 208
_VOCAB = 1000000
_DIM = 64
_NCLASS = 100
_PROJ = 128

_NCHUNK = 13
_CHUNK = 16
_PGRP = _PROJ // 16

_MM_BLK = 2048


def _tc_project(table_t, fc_weight):
  def body(t_ref, w_ref, o_ref):
    acc = lax.dot_general(
        t_ref[:, :], w_ref[:, :],
        dimension_numbers=(((0,), (1,)), ((), ())),
        preferred_element_type=jnp.float32,
    )
    o_ref[:, :] = jnp.pad(acc, ((0, 0), (0, _PROJ - _NCLASS)))

  return pl.pallas_call(
      body,
      grid=((_VOCAB + _MM_BLK - 1) // _MM_BLK,),
      in_specs=[
          pl.BlockSpec((_DIM, _MM_BLK), lambda i: (0, i)),
          pl.BlockSpec((_NCLASS, _DIM), lambda i: (0, 0)),
      ],
      out_specs=pl.BlockSpec((_MM_BLK, _PROJ), lambda i: (i, 0)),
      out_shape=jax.ShapeDtypeStruct((_VOCAB, _PROJ), jnp.float32),
      compiler_params=pltpu.CompilerParams(fuse_transposed_lhs_in_matmul=True),
  )(table_t, fc_weight)


def _sc_gather_sum(idx2, proj):
  mesh = plsc.VectorSubcoreMesh(core_axis_name="c", subcore_axis_name="s")
  nw = mesh.num_cores * mesh.num_subcores
  rows_per_w = _BATCH // nw
  half = rows_per_w // 2

  @functools.partial(
      pl.kernel,
      out_type=jax.ShapeDtypeStruct((_BATCH, _PROJ), jnp.float32),
      mesh=mesh,
      scratch_types=[
          pltpu.VMEM((half, _HISTP), jnp.int32),
          pltpu.VMEM((2, _RING, _PROJ), jnp.float32),
          pltpu.VMEM((rows_per_w, _PROJ), jnp.float32),
          pltpu.SemaphoreType.DMA,
          pltpu.SemaphoreType.DMA,
      ],
      compiler_params=pltpu.CompilerParams(use_tc_tiling_on_sc=True),
  )
  def k(idx_hbm, proj_hbm, out_hbm, idx_v, rows_v, acc_v, sem0, sem1):
    wid = lax.axis_index("s") * mesh.num_cores + lax.axis_index("c")
    base = wid * rows_per_w

    starts = [min(16 * j, _HIST - _CHUNK) for j in range(_NCHUNK)]

    def fire(bh, par, sem):
      for s in starts:
        iv = idx_v[bh, pl.ds(s, _CHUNK)]
        pltpu.async_copy(proj_hbm.at[iv], rows_v.at[par, pl.ds(s, _CHUNK)],
                         sem)

    def drain(bh, par, sem):
      for s in starts:
        iv = idx_v[bh, pl.ds(s, _CHUNK)]
        pltpu.make_async_copy(
            proj_hbm.at[iv], rows_v.at[par, pl.ds(s, _CHUNK)], sem).wait()

    def accumulate_and_store(b, par):
      accs = tuple(jnp.zeros((16,), jnp.float32) for _ in range(_PGRP))

      def body(kk, accs):
        accs = list(accs)
        for r in range(8):
          row = kk * 8 + r
          for g in range(_PGRP):
            accs[g] = accs[g] + rows_v[par, row, pl.ds(g * 16, 16)]
        return tuple(accs)

      accs = lax.fori_loop(0, _HIST // 8, body, accs)
      for g in range(_PGRP):
        acc_v[b, pl.ds(g * 16, 16)] = accs[g]

    for h in range(2):
      hbase = base + h * half
      pltpu.sync_copy(idx_hbm.at[pl.ds(hbase, half)], idx_v)

      fire(0, 0, sem0)

      def step(bb, _, h=h):
        b0 = 2 * bb
        b1 = 2 * bb + 1
        fire(b1, 1, sem1)
        drain(b0, 0, sem0)
        accumulate_and_store(h * half + b0, 0)

        @pl.when(bb < half // 2 - 1)
        def _():
          fire(b0 + 2, 0, sem0)

        drain(b1, 1, sem1)
        accumulate_and_store(h * half + b1, 1)
        return 0

      lax.fori_loop(0, half // 2, step, 0)

    pltpu.sync_copy(acc_v, out_hbm.at[pl.ds(base, rows_per_w)])

  return k(idx2, proj)


def _tc_finish(sums2, fc_bias2):
  def body(x_ref, b_ref, o_ref):
    o_ref[:, :] = x_ref[:, :_NCLASS] * (1.0 / _HIST) + b_ref[:, :]

  return pl.pallas_call(
      body,
      out_shape=jax.ShapeDtypeStruct((_BATCH, _NCLASS), jnp.float32),
  )(sums2, fc_bias2)


def kernel(text, embed_table, fc_weight, fc_bias):
  idx2 = jnp.pad(text.astype(jnp.int32), ((0, 0), (0, _HISTP - _HIST)))
  proj = _tc_project(embed_table.T, fc_weight)
  sums2 = _sc_gather_sum(idx2, proj)
  return _tc_finish(sums2, fc_bias.reshape(1, _NCLASS))
